# baseline (speedup 1.0000x reference)
import jax
import jax.numpy as jnp
from jax import lax
from jax.experimental import pallas as pl
from jax.experimental.pallas import tpu as pltpu
from jax.experimental.pallas import tpu_sc as plsc

N = 10000
E = 320000
D = 128
NC = 2
NS = 16
NW = NC * NS
EPT = E // NW
B = 80
STEPS = EPT // B
RPT = 624
TAIL = N - NS * RPT


def _sc_body(p_hbm, q_hbm, src_hbm, dst_hbm, out_hbm, cnt_hbm,
             src0, dst0, src1, dst1, p0, q0, p1, q1, dsc0, dsc1,
             ones_v, zc_v, cstage_v, s_sh, c_sh,
             sem_p0, sem_q0, sem_p1, sem_q1,
             sem_s0, sem_d0, sem_s1, sem_d1,
             sem_w0, sem_c0, sem_w1, sem_c1):
    cid = lax.axis_index("c")
    sid = lax.axis_index("s")
    wid = sid * NC + cid
    ebase = wid * EPT

    bufs = ((src0, dst0, p0, q0, sem_p0, sem_q0, sem_s0, sem_d0,
             sem_w0, sem_c0, dsc0),
            (src1, dst1, p1, q1, sem_p1, sem_q1, sem_s1, sem_d1,
             sem_w1, sem_c1, dsc1))

    zero = jnp.zeros((16,), jnp.float32)
    one = jnp.ones((16,), jnp.float32)

    def _zr(r, carry):
        for c in range(D // 16):
            q0[r, pl.ds(c * 16, 16)] = zero
        return carry

    lax.fori_loop(0, B, _zr, 0)
    for c in range(B // 16):
        ones_v[pl.ds(c * 16, 16)] = one
        zc_v[pl.ds(c * 16, 16)] = zero

    row0 = sid * RPT
    off = 0
    while off < RPT:
        n = min(B, RPT - off)
        pltpu.sync_copy(q0.at[pl.ds(0, n)], s_sh.at[pl.ds(row0 + off, n)])
        pltpu.sync_copy(zc_v.at[pl.ds(0, n)], c_sh.at[pl.ds(row0 + off, n)])
        off += n

    @pl.when(sid == NS - 1)
    def _zero_tail():
        pltpu.sync_copy(q0.at[pl.ds(0, TAIL)], s_sh.at[pl.ds(NS * RPT, TAIL)])
        pltpu.sync_copy(zc_v.at[pl.ds(0, TAIL)], c_sh.at[pl.ds(NS * RPT, TAIL)])

    plsc.subcore_barrier()

    def _idx_start(j, bf):
        base = ebase + j * B
        pltpu.async_copy(src_hbm.at[pl.ds(base, B)], bf[0], bf[6])
        pltpu.async_copy(dst_hbm.at[pl.ds(base, B)], bf[1], bf[7])

    def _idx_wait(j, bf):
        base = ebase + j * B
        pltpu.make_async_copy(src_hbm.at[pl.ds(base, B)], bf[0], bf[6]).wait()
        pltpu.make_async_copy(dst_hbm.at[pl.ds(base, B)], bf[1], bf[7]).wait()

    def _gather_start(bf):
        pltpu.async_copy(p_hbm.at[bf[0]], bf[2], bf[4])
        pltpu.async_copy(q_hbm.at[bf[1]], bf[3], bf[5])

    def _gather_wait(bf):
        pltpu.make_async_copy(p_hbm.at[bf[0]], bf[2], bf[4]).wait()
        pltpu.make_async_copy(q_hbm.at[bf[1]], bf[3], bf[5]).wait()

    def _compute(bf):
        p_v, q_v = bf[2], bf[3]

        def _row(r, c2):
            for c in range(D // 16):
                sl = pl.ds(c * 16, 16)
                q_v[r, sl] = jnp.maximum(p_v[r, sl] + q_v[r, sl], 0.0)
            return c2

        lax.fori_loop(0, B, _row, 0, unroll=4)

    def _save_idx(bf):
        for c in range(B // 16):
            bf[10][pl.ds(c * 16, 16)] = bf[1][pl.ds(c * 16, 16)]

    def _scatter_start(bf):
        pltpu.async_copy(bf[3], s_sh.at[bf[10]], bf[8], add=True)
        pltpu.async_copy(ones_v, c_sh.at[bf[10]], bf[9], add=True)

    def _scatter_wait(bf):
        pltpu.make_async_copy(bf[3], s_sh.at[bf[10]], bf[8]).wait()
        pltpu.make_async_copy(ones_v, c_sh.at[bf[10]], bf[9]).wait()

    _idx_start(0, bufs[0])
    _idx_wait(0, bufs[0])
    _gather_start(bufs[0])
    _idx_start(1, bufs[1])

    def _pair(i, carry):
        for k in range(2):
            j = 2 * i + k
            cur, nxt = bufs[k], bufs[1 - k]
            _gather_wait(cur)
            _idx_wait(j + 1, nxt)

            @pl.when(j > 0)
            def _drain():
                _scatter_wait(nxt)

            _gather_start(nxt)
            _save_idx(cur)
            _idx_start(j + 2, cur)
            _compute(cur)
            _scatter_start(cur)
        return carry

    lax.fori_loop(0, (STEPS - 1) // 2, _pair, 0)
    _gather_wait(bufs[0])
    _scatter_wait(bufs[1])
    _save_idx(bufs[0])
    _compute(bufs[0])
    _scatter_start(bufs[0])
    _scatter_wait(bufs[0])
    _idx_wait(STEPS, bufs[1])

    plsc.subcore_barrier()
    pltpu.sync_copy(s_sh.at[pl.ds(row0, RPT)],
                    out_hbm.at[pl.ds(cid * N + row0, RPT)])
    pltpu.sync_copy(c_sh.at[pl.ds(row0, RPT)], cstage_v)
    pltpu.sync_copy(cstage_v, cnt_hbm.at[pl.ds(cid * N + row0, RPT)])

    @pl.when(sid == NS - 1)
    def _out_tail():
        pltpu.sync_copy(s_sh.at[pl.ds(NS * RPT, TAIL)],
                        out_hbm.at[pl.ds(cid * N + NS * RPT, TAIL)])
        pltpu.sync_copy(c_sh.at[pl.ds(NS * RPT, TAIL)],
                        cstage_v.at[pl.ds(0, TAIL)])
        pltpu.sync_copy(cstage_v.at[pl.ds(0, TAIL)],
                        cnt_hbm.at[pl.ds(cid * N + NS * RPT, TAIL)])


def _sc_scatter(p, q, src, dst):
    mesh = plsc.VectorSubcoreMesh(core_axis_name="c", subcore_axis_name="s")
    f = pl.kernel(
        _sc_body,
        mesh=mesh,
        out_type=(jax.ShapeDtypeStruct((NC * N, D), jnp.float32),
                  jax.ShapeDtypeStruct((NC * N,), jnp.float32)),
        scratch_types=[
            pltpu.VMEM((B,), jnp.int32),
            pltpu.VMEM((B,), jnp.int32),
            pltpu.VMEM((B,), jnp.int32),
            pltpu.VMEM((B,), jnp.int32),
            pltpu.VMEM((B, D), jnp.float32),
            pltpu.VMEM((B, D), jnp.float32),
            pltpu.VMEM((B, D), jnp.float32),
            pltpu.VMEM((B, D), jnp.float32),
            pltpu.VMEM((B,), jnp.int32),
            pltpu.VMEM((B,), jnp.int32),
            pltpu.VMEM((B,), jnp.float32),
            pltpu.VMEM((B,), jnp.float32),
            pltpu.VMEM((RPT,), jnp.float32),
            pltpu.VMEM_SHARED((N, D), jnp.float32),
            pltpu.VMEM_SHARED((N,), jnp.float32),
        ] + [pltpu.SemaphoreType.DMA] * 12,
    )
    return f(p, q, src, dst)


def _pre_body(x_ref, w1a_ref, w1b_ref, b1_ref, p_ref, q_ref):
    xb = x_ref[...]
    p_ref[...] = jnp.dot(xb, w1a_ref[...], preferred_element_type=jnp.float32)
    q_ref[...] = (jnp.dot(xb, w1b_ref[...], preferred_element_type=jnp.float32)
                  + b1_ref[...])


def _pre(x, w1a_t, w1b_t, b1):
    bn = 2000
    xs = pl.BlockSpec((bn, D), lambda i: (i, 0))
    ws = pl.BlockSpec((D, D), lambda i: (0, 0))
    bs = pl.BlockSpec((1, D), lambda i: (0, 0))
    return pl.pallas_call(
        _pre_body,
        grid=(N // bn,),
        in_specs=[xs, ws, ws, bs],
        out_specs=[xs, xs],
        out_shape=[jax.ShapeDtypeStruct((N, D), jnp.float32)] * 2,
    )(x, w1a_t, w1b_t, b1)


def _post_body(s_ref, c_ref, x_ref, w2t_ref, b2_ref, wst_ref, bs_ref, o_ref):
    sb = s_ref[...]
    s = sb[0] + sb[1]
    cb = c_ref[...]
    cnt = cb[0] + cb[1]
    deg = jnp.maximum(cnt, 1.0)
    m = jnp.dot(s, w2t_ref[...], preferred_element_type=jnp.float32) / deg
    m = m + jnp.where(cnt > 0.0, 1.0, 0.0) * b2_ref[...]
    o_ref[...] = (m + jnp.dot(x_ref[...], wst_ref[...],
                              preferred_element_type=jnp.float32) + bs_ref[...])


def _post(s, c, x, w2t, b2, wst, bsf):
    bn = 2000
    ss = pl.BlockSpec((NC, bn, D), lambda i: (0, i, 0))
    cs = pl.BlockSpec((NC, bn, 1), lambda i: (0, i, 0))
    xs = pl.BlockSpec((bn, D), lambda i: (i, 0))
    ws = pl.BlockSpec((D, D), lambda i: (0, 0))
    bs = pl.BlockSpec((1, D), lambda i: (0, 0))
    return pl.pallas_call(
        _post_body,
        grid=(N // bn,),
        in_specs=[ss, cs, xs, ws, bs, ws, bs],
        out_specs=xs,
        out_shape=jax.ShapeDtypeStruct((N, D), jnp.float32),
    )(s, c, x, w2t, b2, wst, bsf)


@jax.jit
def kernel(x, edge_index, W_msg1, b_msg1, W_msg2, b_msg2, W_self, b_self):
    w1a_t = W_msg1[:, :D].T
    w1b_t = W_msg1[:, D:].T
    p, q = _pre(x, w1a_t, w1b_t, b_msg1.reshape(1, D))
    pad = jnp.zeros((2 * B,), jnp.int32)
    src = jnp.concatenate([edge_index[0], pad])
    dst = jnp.concatenate([edge_index[1], pad])
    s_flat, c_flat = _sc_scatter(p, q, src, dst)
    s = s_flat.reshape(NC, N, D)
    c = c_flat.reshape(NC, N, 1)
    return _post(s, c, x, W_msg2.T, b_msg2.reshape(1, D),
                 W_self.T, b_self.reshape(1, D))


# speedup vs baseline: 1.9899x; 1.9899x over previous
import jax
import jax.numpy as jnp
from jax import lax
from jax.experimental import pallas as pl
from jax.experimental.pallas import tpu as pltpu
from jax.experimental.pallas import tpu_sc as plsc

N = 10000
E = 320000
D = 128
NC = 2
NS = 16
NW = NC * NS
EPT = E // NW
B = 80
STEPS = EPT // B
RPT = 624
TAIL = N - NS * RPT


def _sc_body(p_hbm, q_hbm, src_hbm, dst_hbm, out_hbm, cnt_hbm,
             src0, dst0, src1, dst1, p0, q0, p1, q1, dsc0, dsc1,
             ones_v, zc_v, cstage_v, s_sh, c_sh,
             sem_p0, sem_q0, sem_p1, sem_q1,
             sem_s0, sem_d0, sem_s1, sem_d1,
             sem_w0, sem_c0, sem_w1, sem_c1):
    cid = lax.axis_index("c")
    sid = lax.axis_index("s")
    wid = sid * NC + cid
    ebase = wid * EPT

    bufs = ((src0, dst0, p0, q0, sem_p0, sem_q0, sem_s0, sem_d0,
             sem_w0, sem_c0, dsc0),
            (src1, dst1, p1, q1, sem_p1, sem_q1, sem_s1, sem_d1,
             sem_w1, sem_c1, dsc1))

    zero = jnp.zeros((16,), jnp.float32)
    one = jnp.ones((16,), jnp.float32)

    def _zr(r, carry):
        for c in range(D // 16):
            q0[r, pl.ds(c * 16, 16)] = zero
        return carry

    lax.fori_loop(0, B, _zr, 0)
    for c in range(B // 16):
        ones_v[pl.ds(c * 16, 16)] = one
        zc_v[pl.ds(c * 16, 16)] = zero

    row0 = sid * RPT
    off = 0
    while off < RPT:
        n = min(B, RPT - off)
        pltpu.sync_copy(q0.at[pl.ds(0, n)], s_sh.at[pl.ds(row0 + off, n)])
        pltpu.sync_copy(zc_v.at[pl.ds(0, n)], c_sh.at[pl.ds(row0 + off, n)])
        off += n

    @pl.when(sid == NS - 1)
    def _zero_tail():
        pltpu.sync_copy(q0.at[pl.ds(0, TAIL)], s_sh.at[pl.ds(NS * RPT, TAIL)])
        pltpu.sync_copy(zc_v.at[pl.ds(0, TAIL)], c_sh.at[pl.ds(NS * RPT, TAIL)])

    plsc.subcore_barrier()

    def _idx_start(j, bf):
        base = ebase + j * B
        pltpu.async_copy(src_hbm.at[pl.ds(base, B)], bf[0], bf[6])
        pltpu.async_copy(dst_hbm.at[pl.ds(base, B)], bf[1], bf[7])

    def _idx_wait(j, bf):
        base = ebase + j * B
        pltpu.make_async_copy(src_hbm.at[pl.ds(base, B)], bf[0], bf[6]).wait()
        pltpu.make_async_copy(dst_hbm.at[pl.ds(base, B)], bf[1], bf[7]).wait()

    def _gather_start(bf):
        pltpu.async_copy(p_hbm.at[bf[0]], bf[2], bf[4])
        pltpu.async_copy(q_hbm.at[bf[1]], bf[3], bf[5])

    def _gather_wait(bf):
        pltpu.make_async_copy(p_hbm.at[bf[0]], bf[2], bf[4]).wait()
        pltpu.make_async_copy(q_hbm.at[bf[1]], bf[3], bf[5]).wait()

    def _compute(bf):
        p_v, q_v = bf[2], bf[3]

        def _row(r, c2):
            for c in range(D // 16):
                sl = pl.ds(c * 16, 16)
                q_v[r, sl] = jnp.maximum(p_v[r, sl] + q_v[r, sl], 0.0)
            return c2

        lax.fori_loop(0, B, _row, 0)

    def _save_idx(bf):
        for c in range(B // 16):
            bf[10][pl.ds(c * 16, 16)] = bf[1][pl.ds(c * 16, 16)]

    def _scatter_start(bf):
        pltpu.async_copy(bf[3], s_sh.at[bf[10]], bf[8], add=True)
        pltpu.async_copy(ones_v, c_sh.at[bf[10]], bf[9], add=True)

    def _scatter_wait(bf):
        pltpu.make_async_copy(bf[3], s_sh.at[bf[10]], bf[8]).wait()
        pltpu.make_async_copy(ones_v, c_sh.at[bf[10]], bf[9]).wait()

    _idx_start(0, bufs[0])
    _idx_wait(0, bufs[0])
    _gather_start(bufs[0])
    _idx_start(1, bufs[1])

    def _pair(i, carry):
        for k in range(2):
            j = 2 * i + k
            cur, nxt = bufs[k], bufs[1 - k]
            _gather_wait(cur)
            _idx_wait(j + 1, nxt)

            @pl.when(j > 0)
            def _drain():
                _scatter_wait(nxt)

            _gather_start(nxt)
            _save_idx(cur)
            _idx_start(j + 2, cur)
            _compute(cur)
            _scatter_start(cur)
        return carry

    lax.fori_loop(0, (STEPS - 1) // 2, _pair, 0)
    _gather_wait(bufs[0])
    _scatter_wait(bufs[1])
    _save_idx(bufs[0])
    _compute(bufs[0])
    _scatter_start(bufs[0])
    _scatter_wait(bufs[0])
    _idx_wait(STEPS, bufs[1])

    plsc.subcore_barrier()
    pltpu.sync_copy(s_sh.at[pl.ds(row0, RPT)],
                    out_hbm.at[pl.ds(cid * N + row0, RPT)])
    pltpu.sync_copy(c_sh.at[pl.ds(row0, RPT)], cstage_v)
    pltpu.sync_copy(cstage_v, cnt_hbm.at[pl.ds(cid * N + row0, RPT)])

    @pl.when(sid == NS - 1)
    def _out_tail():
        pltpu.sync_copy(s_sh.at[pl.ds(NS * RPT, TAIL)],
                        out_hbm.at[pl.ds(cid * N + NS * RPT, TAIL)])
        pltpu.sync_copy(c_sh.at[pl.ds(NS * RPT, TAIL)],
                        cstage_v.at[pl.ds(0, TAIL)])
        pltpu.sync_copy(cstage_v.at[pl.ds(0, TAIL)],
                        cnt_hbm.at[pl.ds(cid * N + NS * RPT, TAIL)])


def _sc_scatter(p, q, src, dst):
    mesh = plsc.VectorSubcoreMesh(core_axis_name="c", subcore_axis_name="s")
    f = pl.kernel(
        _sc_body,
        mesh=mesh,
        out_type=(jax.ShapeDtypeStruct((NC * N, D), jnp.float32),
                  jax.ShapeDtypeStruct((NC * N,), jnp.float32)),
        scratch_types=[
            pltpu.VMEM((B,), jnp.int32),
            pltpu.VMEM((B,), jnp.int32),
            pltpu.VMEM((B,), jnp.int32),
            pltpu.VMEM((B,), jnp.int32),
            pltpu.VMEM((B, D), jnp.float32),
            pltpu.VMEM((B, D), jnp.float32),
            pltpu.VMEM((B, D), jnp.float32),
            pltpu.VMEM((B, D), jnp.float32),
            pltpu.VMEM((B,), jnp.int32),
            pltpu.VMEM((B,), jnp.int32),
            pltpu.VMEM((B,), jnp.float32),
            pltpu.VMEM((B,), jnp.float32),
            pltpu.VMEM((RPT,), jnp.float32),
            pltpu.VMEM_SHARED((N, D), jnp.float32),
            pltpu.VMEM_SHARED((N,), jnp.float32),
        ] + [pltpu.SemaphoreType.DMA] * 12,
    )
    return f(p, q, src, dst)


def _pre_body(x_ref, w1a_ref, w1b_ref, b1_ref, p_ref, q_ref):
    xb = x_ref[...]
    p_ref[...] = jnp.dot(xb, w1a_ref[...], preferred_element_type=jnp.float32)
    q_ref[...] = (jnp.dot(xb, w1b_ref[...], preferred_element_type=jnp.float32)
                  + b1_ref[...])


def _pre(x, w1a_t, w1b_t, b1):
    bn = 2000
    xs = pl.BlockSpec((bn, D), lambda i: (i, 0))
    ws = pl.BlockSpec((D, D), lambda i: (0, 0))
    bs = pl.BlockSpec((1, D), lambda i: (0, 0))
    return pl.pallas_call(
        _pre_body,
        grid=(N // bn,),
        in_specs=[xs, ws, ws, bs],
        out_specs=[xs, xs],
        out_shape=[jax.ShapeDtypeStruct((N, D), jnp.float32)] * 2,
    )(x, w1a_t, w1b_t, b1)


def _post_body(s_ref, c_ref, x_ref, w2t_ref, b2_ref, wst_ref, bs_ref, o_ref):
    sb = s_ref[...]
    s = sb[0] + sb[1]
    cb = c_ref[...]
    cnt = cb[0] + cb[1]
    deg = jnp.maximum(cnt, 1.0)
    m = jnp.dot(s, w2t_ref[...], preferred_element_type=jnp.float32) / deg
    m = m + jnp.where(cnt > 0.0, 1.0, 0.0) * b2_ref[...]
    o_ref[...] = (m + jnp.dot(x_ref[...], wst_ref[...],
                              preferred_element_type=jnp.float32) + bs_ref[...])


def _post(s, c, x, w2t, b2, wst, bsf):
    bn = 2000
    ss = pl.BlockSpec((NC, bn, D), lambda i: (0, i, 0))
    cs = pl.BlockSpec((NC, bn, 1), lambda i: (0, i, 0))
    xs = pl.BlockSpec((bn, D), lambda i: (i, 0))
    ws = pl.BlockSpec((D, D), lambda i: (0, 0))
    bs = pl.BlockSpec((1, D), lambda i: (0, 0))
    return pl.pallas_call(
        _post_body,
        grid=(N // bn,),
        in_specs=[ss, cs, xs, ws, bs, ws, bs],
        out_specs=xs,
        out_shape=jax.ShapeDtypeStruct((N, D), jnp.float32),
    )(s, c, x, w2t, b2, wst, bsf)


@jax.jit
def kernel(x, edge_index, W_msg1, b_msg1, W_msg2, b_msg2, W_self, b_self):
    w1a_t = W_msg1[:, :D].T
    w1b_t = W_msg1[:, D:].T
    p, q = _pre(x, w1a_t, w1b_t, b_msg1.reshape(1, D))
    pad = jnp.zeros((2 * B,), jnp.int32)
    src = jnp.concatenate([edge_index[0], pad])
    dst = jnp.concatenate([edge_index[1], pad])
    s_flat, c_flat = _sc_scatter(p, q, src, dst)
    s = s_flat.reshape(NC, N, D)
    c = c_flat.reshape(NC, N, 1)
    return _post(s, c, x, W_msg2.T, b_msg2.reshape(1, D),
                 W_self.T, b_self.reshape(1, D))
